# baseline (speedup 1.0000x reference)
import functools

import jax
import jax.numpy as jnp
from jax import lax
from jax.experimental import pallas as pl
from jax.experimental.pallas import tpu as pltpu
from jax.experimental.pallas import tpu_sc as plsc

_LANES = 16
_NBITS = 30


def _meter_sc(preds_t, targets_t, num_workers, rows_per_worker, C):
    n_groups = rows_per_worker // _LANES

    def body(p_hbm, t_hbm, cor_out, tot_out, p_v, t_v, accc, acct):
        wid = lax.axis_index("s") * 2 + lax.axis_index("c")
        pltpu.sync_copy(p_hbm.at[wid], p_v)
        pltpu.sync_copy(t_hbm.at[wid], t_v)

        zero = jnp.zeros((_LANES,), jnp.float32)
        for j in range(C):
            accc[pl.ds(j * _LANES, _LANES)] = zero
            acct[pl.ds(j * _LANES, _LANES)] = zero

        def group(g, _):
            off = g * _LANES
            ksums = [zero, zero, zero, zero]
            for j in range(C):
                tv = t_v[pl.ds(j * rows_per_worker + off, _LANES)]
                ksums[j % 4] = ksums[j % 4] + tv
                ja = pl.ds(j * _LANES, _LANES)
                acct[ja] = acct[ja] + tv
            ksum = (ksums[0] + ksums[1]) + (ksums[2] + ksums[3])
            k = jnp.clip(ksum.astype(jnp.int32), 1, C)

            izero = jnp.zeros((_LANES,), jnp.int32)

            def bit_round(i, cur):
                cand = cur | (jnp.int32(1) << (jnp.int32(_NBITS - 1) - i))
                cnts = [izero, izero, izero, izero]
                for j in range(C):
                    pi = p_v[pl.ds(j * rows_per_worker + off, _LANES)]
                    cnts[j % 4] = cnts[j % 4] + jnp.where(pi >= cand, 1, 0)
                cnt = (cnts[0] + cnts[1]) + (cnts[2] + cnts[3])
                return jnp.where(cnt >= k, cand, cur)

            kth = lax.fori_loop(0, _NBITS, bit_round, jnp.zeros((_LANES,), jnp.int32))

            for j in range(C):
                pi = p_v[pl.ds(j * rows_per_worker + off, _LANES)]
                tv = t_v[pl.ds(j * rows_per_worker + off, _LANES)]
                ja = pl.ds(j * _LANES, _LANES)
                accc[ja] = accc[ja] + jnp.where(pi >= kth, tv, 0.0)
            return ()

        lax.fori_loop(0, n_groups, group, ())

        pltpu.sync_copy(accc, cor_out.at[wid])
        pltpu.sync_copy(acct, tot_out.at[wid])

    return pl.kernel(
        body,
        out_type=(
            jax.ShapeDtypeStruct((num_workers, C * _LANES), jnp.float32),
            jax.ShapeDtypeStruct((num_workers, C * _LANES), jnp.float32),
        ),
        mesh=plsc.VectorSubcoreMesh(core_axis_name="c", subcore_axis_name="s"),
        scratch_types=[
            pltpu.VMEM((C * rows_per_worker,), jnp.int32),
            pltpu.VMEM((C * rows_per_worker,), jnp.float32),
            pltpu.VMEM((C * _LANES,), jnp.float32),
            pltpu.VMEM((C * _LANES,), jnp.float32),
        ],
    )(preds_t, targets_t)


@jax.jit
def kernel(preds, targets, corrects, totals):
    N, C = preds.shape
    info = plsc.get_sparse_core_info()
    num_workers = info.num_cores * info.num_subcores
    rows_per_worker = N // num_workers
    preds_i = lax.bitcast_convert_type(preds.T, jnp.int32)
    preds_w = preds_i.reshape(C, num_workers, rows_per_worker).transpose(1, 0, 2).reshape(num_workers, C * rows_per_worker)
    targets_w = targets.T.reshape(C, num_workers, rows_per_worker).transpose(1, 0, 2).reshape(num_workers, C * rows_per_worker)
    cor_p, tot_p = _meter_sc(preds_w, targets_w, num_workers, rows_per_worker, C)
    cb = cor_p.reshape(num_workers, C, _LANES).sum(axis=(0, 2))
    tb = tot_p.reshape(num_workers, C, _LANES).sum(axis=(0, 2))
    return (corrects + cb, totals + tb, cb.sum(), tb.sum())


# speedup vs baseline: 1.1002x; 1.1002x over previous
import functools

import jax
import jax.numpy as jnp
from jax import lax
from jax.experimental import pallas as pl
from jax.experimental.pallas import tpu as pltpu
from jax.experimental.pallas import tpu_sc as plsc

_LANES = 16
_NBITS = 30


def _meter_sc(preds_t, targets_t, num_workers, rows_per_worker):
    C, N = preds_t.shape
    n_groups = rows_per_worker // _LANES

    def body(p_hbm, t_hbm, cor_out, tot_out, p_v, t_v, accc, acct):
        wid = lax.axis_index("s") * 2 + lax.axis_index("c")
        base = wid * rows_per_worker
        pltpu.sync_copy(p_hbm.at[:, pl.ds(base, rows_per_worker)], p_v)
        pltpu.sync_copy(t_hbm.at[:, pl.ds(base, rows_per_worker)], t_v)

        zero = jnp.zeros((_LANES,), jnp.float32)
        for j in range(C):
            accc[pl.ds(j * _LANES, _LANES)] = zero
            acct[pl.ds(j * _LANES, _LANES)] = zero

        def group(g, _):
            off = g * _LANES
            ksums = [zero, zero, zero, zero]
            for j in range(C):
                tv = t_v[j, pl.ds(off, _LANES)]
                ksums[j % 4] = ksums[j % 4] + tv
                ja = pl.ds(j * _LANES, _LANES)
                acct[ja] = acct[ja] + tv
            ksum = (ksums[0] + ksums[1]) + (ksums[2] + ksums[3])
            k = jnp.clip(ksum.astype(jnp.int32), 1, C)

            izero = jnp.zeros((_LANES,), jnp.int32)

            km = k - jnp.int32(C)

            def bit_round(i, cur):
                cand = cur | (jnp.int32(1) << (jnp.int32(_NBITS - 1) - i))
                cnts = [izero, izero, izero, izero]
                for j in range(C):
                    pi = p_v[j, pl.ds(off, _LANES)]
                    cnts[j % 4] = cnts[j % 4] + ((pi - cand) >> 31)
                cnt = (cnts[0] + cnts[1]) + (cnts[2] + cnts[3])
                return jnp.where(cnt >= km, cand, cur)

            kth = lax.fori_loop(0, _NBITS, bit_round, jnp.zeros((_LANES,), jnp.int32))

            for j in range(C):
                pi = p_v[j, pl.ds(off, _LANES)]
                tv = t_v[j, pl.ds(off, _LANES)]
                ja = pl.ds(j * _LANES, _LANES)
                accc[ja] = accc[ja] + jnp.where(pi >= kth, tv, 0.0)
            return ()

        lax.fori_loop(0, n_groups, group, ())

        pltpu.sync_copy(accc, cor_out.at[wid])
        pltpu.sync_copy(acct, tot_out.at[wid])

    return pl.kernel(
        body,
        out_type=(
            jax.ShapeDtypeStruct((num_workers, C * _LANES), jnp.float32),
            jax.ShapeDtypeStruct((num_workers, C * _LANES), jnp.float32),
        ),
        mesh=plsc.VectorSubcoreMesh(core_axis_name="c", subcore_axis_name="s"),
        scratch_types=[
            pltpu.VMEM((C, rows_per_worker), jnp.int32),
            pltpu.VMEM((C, rows_per_worker), jnp.float32),
            pltpu.VMEM((C * _LANES,), jnp.float32),
            pltpu.VMEM((C * _LANES,), jnp.float32),
        ],
    )(preds_t, targets_t)


@jax.jit
def kernel(preds, targets, corrects, totals):
    N, C = preds.shape
    info = plsc.get_sparse_core_info()
    num_workers = info.num_cores * info.num_subcores
    rows_per_worker = N // num_workers
    preds_i = lax.bitcast_convert_type(preds.T, jnp.int32)
    cor_p, tot_p = _meter_sc(preds_i, targets.T, num_workers, rows_per_worker)
    cb = cor_p.reshape(num_workers, C, _LANES).sum(axis=(0, 2))
    tb = tot_p.reshape(num_workers, C, _LANES).sum(axis=(0, 2))
    return (corrects + cb, totals + tb, cb.sum(), tb.sum())


# speedup vs baseline: 1.2562x; 1.1418x over previous
import functools

import jax
import jax.numpy as jnp
from jax import lax
from jax.experimental import pallas as pl
from jax.experimental.pallas import tpu as pltpu
from jax.experimental.pallas import tpu_sc as plsc

_LANES = 16
_NBITS = 30


def _meter_sc(preds_t, targets_t, num_workers, rows_per_worker):
    C, N = preds_t.shape
    n_groups = rows_per_worker // _LANES

    def body(p_hbm, t_hbm, cor_out, tot_out, p_v, t_v, accc, acct):
        wid = lax.axis_index("s") * 2 + lax.axis_index("c")
        base = wid * rows_per_worker
        pltpu.sync_copy(p_hbm.at[:, pl.ds(base, rows_per_worker)], p_v)
        pltpu.sync_copy(t_hbm.at[:, pl.ds(base, rows_per_worker)], t_v)

        zero = jnp.zeros((_LANES,), jnp.float32)
        for j in range(C):
            accc[pl.ds(j * _LANES, _LANES)] = zero
            acct[pl.ds(j * _LANES, _LANES)] = zero

        def group(g, _):
            off = g * _LANES
            ksum = zero
            for j in range(C):
                tv = t_v[j, pl.ds(off, _LANES)]
                ksum = ksum + tv
                ja = pl.ds(j * _LANES, _LANES)
                acct[ja] = acct[ja] + tv
            k = jnp.clip(ksum.astype(jnp.int32), 1, C)

            izero = jnp.zeros((_LANES,), jnp.int32)

            def bit_round(i, cur):
                cand = cur | (jnp.int32(1) << (jnp.int32(_NBITS - 1) - i))
                cnt = izero
                for j in range(C):
                    pi = p_v[j, pl.ds(off, _LANES)]
                    cnt = cnt + jnp.where(pi >= cand, 1, 0)
                return jnp.where(cnt >= k, cand, cur)

            kth = lax.fori_loop(0, _NBITS, bit_round, jnp.zeros((_LANES,), jnp.int32))

            for j in range(C):
                pi = p_v[j, pl.ds(off, _LANES)]
                tv = t_v[j, pl.ds(off, _LANES)]
                ja = pl.ds(j * _LANES, _LANES)
                accc[ja] = accc[ja] + jnp.where(pi >= kth, tv, 0.0)
            return ()

        lax.fori_loop(0, n_groups, group, ())

        pltpu.sync_copy(accc, cor_out.at[wid])
        pltpu.sync_copy(acct, tot_out.at[wid])

    return pl.kernel(
        body,
        out_type=(
            jax.ShapeDtypeStruct((num_workers, C * _LANES), jnp.float32),
            jax.ShapeDtypeStruct((num_workers, C * _LANES), jnp.float32),
        ),
        mesh=plsc.VectorSubcoreMesh(core_axis_name="c", subcore_axis_name="s"),
        scratch_types=[
            pltpu.VMEM((C, rows_per_worker), jnp.int32),
            pltpu.VMEM((C, rows_per_worker), jnp.float32),
            pltpu.VMEM((C * _LANES,), jnp.float32),
            pltpu.VMEM((C * _LANES,), jnp.float32),
        ],
    )(preds_t, targets_t)


@jax.jit
def kernel(preds, targets, corrects, totals):
    N, C = preds.shape
    info = plsc.get_sparse_core_info()
    num_workers = info.num_cores * info.num_subcores
    rows_per_worker = N // num_workers
    preds_i = lax.bitcast_convert_type(preds.T, jnp.int32)
    cor_p, tot_p = _meter_sc(preds_i, targets.T, num_workers, rows_per_worker)
    cb = cor_p.reshape(num_workers, C, _LANES).sum(axis=(0, 2))
    tb = tot_p.reshape(num_workers, C, _LANES).sum(axis=(0, 2))
    return (corrects + cb, totals + tb, cb.sum(), tb.sum())
